# baseline (speedup 1.0000x reference)
import jax
import jax.numpy as jnp
from jax import lax
from jax.experimental import pallas as pl
from jax.experimental.pallas import tpu as pltpu
from jax.experimental.pallas import tpu_sc as plsc

D = 64
DP = 128
NC = 2
NS = 16
NW = NC * NS
NBUF = 8
LOOKA = 5
NTF = 7812
JPW = 245


def _tr_body(tT_hbm, tail_hbm, outp_hbm, a_v, t_v, isem, osem):
    wid = lax.axis_index("s") * NC + lax.axis_index("c")
    iota = lax.broadcasted_iota(jnp.int32, (16,), 0)

    def in_start(j, ab):
        for i in range(8):
            pltpu.make_async_copy(
                tT_hbm.at[pl.ds(8 * i, 8), pl.ds(128 * j, 128)],
                a_v.at[ab, i], isem.at[ab],
            ).start()

    def in_wait(j, ab):
        for i in range(8):
            pltpu.make_async_copy(
                tT_hbm.at[pl.ds(8 * i, 8), pl.ds(128 * j, 128)],
                a_v.at[ab, i], isem.at[ab],
            ).wait()

    def out_start(j, tb):
        pltpu.make_async_copy(
            t_v.at[tb], outp_hbm.at[pl.ds(128 * j, 128)], osem.at[tb]
        ).start()

    def out_wait(j, tb):
        pltpu.make_async_copy(
            t_v.at[tb], outp_hbm.at[pl.ds(128 * j, 128)], osem.at[tb]
        ).wait()

    def jof(jj):
        return jj * NW + wid

    @pl.when(jof(0) < NTF)
    def _prime():
        in_start(jof(0), 0)

    def step(jj, carry):
        j = jof(jj)
        ab = jj % 2

        @pl.when(jnp.logical_and(jj >= 2, jof(jj - 2) < NTF))
        def _drain_out():
            out_wait(jof(jj - 2), ab)

        @pl.when(j < NTF)
        def _do():
            in_wait(j, ab)

            @pl.when(jof(jj + 1) < NTF)
            def _next_in():
                in_start(jof(jj + 1), 1 - ab)

            def v0i_body(v0i, c2):
                v0 = 16 * v0i
                for i in range(8):
                    for r in range(8):
                        d = 8 * i + r
                        vals = a_v[ab, i, r, pl.ds(v0, 16)]
                        plsc.store_scatter(
                            t_v.at[ab],
                            [v0 + iota, jnp.full((16,), d, jnp.int32)],
                            vals,
                        )
                return c2

            lax.fori_loop(0, 8, v0i_body, 0, unroll=False)
            out_start(j, ab)

        return carry

    lax.fori_loop(0, JPW, step, 0, unroll=False)

    for k in (JPW - 2, JPW - 1):
        @pl.when(jof(k) < NTF)
        def _final_drain(k=k):
            out_wait(jof(k), k % 2)

    @pl.when(wid == 0)
    def _tail():
        pltpu.make_async_copy(
            tail_hbm, t_v.at[0, pl.ds(0, 64)], isem.at[0]
        ).start()
        pltpu.make_async_copy(
            tail_hbm, t_v.at[0, pl.ds(0, 64)], isem.at[0]
        ).wait()
        pltpu.make_async_copy(
            t_v.at[0, pl.ds(0, 64)],
            outp_hbm.at[pl.ds(128 * NTF, 64)],
            osem.at[0],
        ).start()
        pltpu.make_async_copy(
            t_v.at[0, pl.ds(0, 64)],
            outp_hbm.at[pl.ds(128 * NTF, 64)],
            osem.at[0],
        ).wait()


def _emb_body(x_hbm, table_hbm, out_hbm, idx_v, rows_v, gsem, wsem):
    nch = x_hbm.shape[0] // NW
    S = x_hbm.shape[1]
    wid = lax.axis_index("s") * NC + lax.axis_index("c")
    base = wid * nch

    pltpu.sync_copy(x_hbm.at[pl.ds(base, nch)], idx_v)

    def gather_start(c, b):
        pltpu.make_async_copy(
            table_hbm.at[idx_v.at[c]], rows_v.at[b], gsem.at[b]
        ).start()

    def gather_wait(c, b):
        pltpu.make_async_copy(
            table_hbm.at[idx_v.at[c]], rows_v.at[b], gsem.at[b]
        ).wait()

    def write_start(c, b):
        pltpu.make_async_copy(
            rows_v.at[b],
            out_hbm.at[pl.ds((base + c) * S, S), pl.ds(0, D)],
            wsem.at[b],
        ).start()

    def write_wait(c, b):
        pltpu.make_async_copy(
            rows_v.at[b],
            out_hbm.at[pl.ds((base + c) * S, S), pl.ds(0, D)],
            wsem.at[b],
        ).wait()

    for c in range(LOOKA):
        gather_start(c, c % NBUF)

    n_groups = nch // NBUF

    def group(g, carry):
        for u in range(NBUF):
            c = g * NBUF + u
            b = u
            gather_wait(c, b)
            write_start(c, b)
            q = c + LOOKA
            bq = (u + LOOKA) % NBUF

            @pl.when(q < nch)
            def _arm_next():
                @pl.when(q >= NBUF)
                def _drain_old_write():
                    write_wait(q - NBUF, bq)

                gather_start(q, bq)

        return carry

    lax.fori_loop(0, n_groups, group, 0, unroll=False)

    for c in range(nch - NBUF, nch):
        write_wait(c, c % NBUF)


def kernel(x, table):
    B, S = x.shape
    V = table.shape[0]
    nch = B // NW
    idx = x.astype(jnp.int32) * 2

    mesh = plsc.VectorSubcoreMesh(
        core_axis_name="c", subcore_axis_name="s",
        num_cores=NC, num_subcores=NS,
    )

    tr = pl.kernel(
        _tr_body,
        out_type=jax.ShapeDtypeStruct((V, DP), jnp.float32),
        mesh=mesh,
        scratch_types=[
            pltpu.VMEM((2, 8, 8, 128), jnp.float32),
            pltpu.VMEM((2, 128, 128), jnp.float32),
            pltpu.SemaphoreType.DMA((2,)),
            pltpu.SemaphoreType.DMA((2,)),
        ],
        compiler_params=pltpu.CompilerParams(
            use_tc_tiling_on_sc=True, needs_layout_passes=False,
        ),
    )
    tail = jnp.pad(table[128 * NTF:, :], ((0, 0), (0, DP - D)))
    tablep = tr(table.T, tail)
    tablep2 = tablep.reshape(2 * V, D)

    emb = pl.kernel(
        _emb_body,
        out_type=jax.ShapeDtypeStruct((B * S, DP), jnp.float32),
        mesh=mesh,
        scratch_types=[
            pltpu.VMEM((nch, S), jnp.int32),
            pltpu.VMEM((NBUF, S, D), jnp.float32),
            pltpu.SemaphoreType.DMA((NBUF,)),
            pltpu.SemaphoreType.DMA((NBUF,)),
        ],
        compiler_params=pltpu.CompilerParams(use_tc_tiling_on_sc=False),
    )
    outp = emb(idx, tablep2)
    return outp[:, :D].reshape(B, S, D)


# speedup vs baseline: 1.7953x; 1.7953x over previous
import jax
import jax.numpy as jnp
from jax import lax
from jax.experimental import pallas as pl
from jax.experimental.pallas import tpu as pltpu
from jax.experimental.pallas import tpu_sc as plsc

D = 64
DP = 128
NC = 2
NS = 16
NW = NC * NS
NBUF = 8
LOOKA = 5


def _emb_body(x_hbm, table_hbm, out_hbm, idx_v, rows_v, gsem, wsem):
    nch = x_hbm.shape[0] // NW
    S = x_hbm.shape[1]
    wid = lax.axis_index("s") * NC + lax.axis_index("c")
    base = wid * nch

    pltpu.sync_copy(x_hbm.at[pl.ds(base, nch)], idx_v)

    def gather_start(c, b):
        pltpu.make_async_copy(
            table_hbm.at[idx_v.at[c]], rows_v.at[b], gsem.at[b]
        ).start()

    def gather_wait(c, b):
        pltpu.make_async_copy(
            table_hbm.at[idx_v.at[c]], rows_v.at[b], gsem.at[b]
        ).wait()

    def write_start(c, b):
        pltpu.make_async_copy(
            rows_v.at[b],
            out_hbm.at[pl.ds((base + c) * S, S), pl.ds(0, D)],
            wsem.at[b],
        ).start()

    def write_wait(c, b):
        pltpu.make_async_copy(
            rows_v.at[b],
            out_hbm.at[pl.ds((base + c) * S, S), pl.ds(0, D)],
            wsem.at[b],
        ).wait()

    for c in range(LOOKA):
        gather_start(c, c % NBUF)

    n_groups = nch // NBUF

    def group(g, carry):
        for u in range(NBUF):
            c = g * NBUF + u
            b = u
            gather_wait(c, b)
            write_start(c, b)
            q = c + LOOKA
            bq = (u + LOOKA) % NBUF

            @pl.when(q < nch)
            def _arm_next():
                @pl.when(q >= NBUF)
                def _drain_old_write():
                    write_wait(q - NBUF, bq)

                gather_start(q, bq)

        return carry

    lax.fori_loop(0, n_groups, group, 0, unroll=False)

    for c in range(nch - NBUF, nch):
        write_wait(c, c % NBUF)


def kernel(x, table):
    B, S = x.shape
    nch = B // NW
    idx = x.astype(jnp.int32) * 2
    tablep = jnp.pad(table, ((0, 0), (0, DP - D)))
    tablep2 = tablep.reshape(2 * tablep.shape[0], D)

    emb = pl.kernel(
        _emb_body,
        out_type=jax.ShapeDtypeStruct((B * S, DP), jnp.float32),
        mesh=plsc.VectorSubcoreMesh(
            core_axis_name="c", subcore_axis_name="s",
            num_cores=NC, num_subcores=NS,
        ),
        scratch_types=[
            pltpu.VMEM((nch, S), jnp.int32),
            pltpu.VMEM((NBUF, S, D), jnp.float32),
            pltpu.SemaphoreType.DMA((NBUF,)),
            pltpu.SemaphoreType.DMA((NBUF,)),
        ],
        compiler_params=pltpu.CompilerParams(use_tc_tiling_on_sc=False),
    )
    outp = emb(idx, tablep2)
    return outp[:, :D].reshape(B, S, D)
